# initial kernel scaffold (unmeasured)
import jax
import jax.numpy as jnp
from jax import lax
from jax.experimental import pallas as pl
from jax.experimental.pallas import tpu as pltpu


def kernel(
    x,
):
    def body(*refs):
        pass

    out_shape = jax.ShapeDtypeStruct(..., jnp.float32)
    return pl.pallas_call(body, out_shape=out_shape)(...)


# baseline (device time: 991727 ns/iter reference)
import jax
import jax.numpy as jnp
from jax import lax
from jax.experimental import pallas as pl
from jax.experimental.pallas import tpu as pltpu

M_SHARD = 32768
M_HALF = M_SHARD // 2
N = 1024

CH = 1024
NC = M_HALF // CH
CPC = 2048
NCP = M_SHARD // CPC


def kernel(x):
    m_shard, n = x.shape
    assert (m_shard, n) == (M_SHARD, N), (m_shard, n)

    def body(x_ref, out_ref, vbuf, lsems, ssems, xs, xr, zs, zr):
        my_x = lax.axis_index("x")
        my_y = lax.axis_index("y")
        my_z = lax.axis_index("z")
        xbar = 1 - my_x
        zbar = 1 - my_z

        barrier_sem = pltpu.get_barrier_semaphore()
        pl.semaphore_signal(
            barrier_sem, inc=1,
            device_id=(xbar, my_y, my_z), device_id_type=pl.DeviceIdType.MESH,
        )
        pl.semaphore_signal(
            barrier_sem, inc=1,
            device_id=(my_x, my_y, zbar), device_id_type=pl.DeviceIdType.MESH,
        )
        pl.semaphore_wait(barrier_sem, 2)

        send_base = my_x * M_SHARD + my_z * M_HALF
        recv_base = xbar * M_SHARD + my_z * M_HALF

        def x_rdma(c):
            return pltpu.make_async_remote_copy(
                src_ref=x_ref.at[pl.ds(my_z * M_HALF + c * CH, CH)],
                dst_ref=out_ref.at[pl.ds(send_base + c * CH, CH)],
                send_sem=xs.at[c], recv_sem=xr.at[c],
                device_id=(xbar, my_y, my_z),
                device_id_type=pl.DeviceIdType.MESH,
            )

        def z_rdma(c):
            return pltpu.make_async_remote_copy(
                src_ref=out_ref.at[pl.ds(recv_base + c * CH, CH)],
                dst_ref=out_ref.at[pl.ds(recv_base + c * CH, CH)],
                send_sem=zs.at[c], recv_sem=zr.at[c],
                device_id=(my_x, my_y, zbar),
                device_id_type=pl.DeviceIdType.MESH,
            )

        for c in range(NC):
            x_rdma(c).start()

        def load(c, slot):
            return pltpu.make_async_copy(
                x_ref.at[pl.ds(c * CPC, CPC)], vbuf.at[slot], lsems.at[slot])

        def store(c, slot):
            return pltpu.make_async_copy(
                vbuf.at[slot], out_ref.at[pl.ds(my_x * M_SHARD + c * CPC, CPC)],
                ssems.at[slot])

        load(0, 0).start()
        for c in range(NCP):
            slot = c % 2
            if c + 1 < NCP:
                if c >= 1:
                    store(c - 1, (c + 1) % 2).wait()
                load(c + 1, (c + 1) % 2).start()
            load(c, slot).wait()
            store(c, slot).start()
        for c in range(max(NCP - 2, 0), NCP):
            store(c, c % 2).wait()

        for c in range(NC):
            x_rdma(c).wait_recv()
            z_rdma(c).start()

        for c in range(NC):
            z_rdma(c).wait_recv()
        for c in range(NC):
            x_rdma(c).wait_send()
            z_rdma(c).wait_send()

    out_shape = jax.ShapeDtypeStruct((2 * M_SHARD, N), jnp.float32)
    return pl.pallas_call(
        body,
        out_shape=out_shape,
        in_specs=[pl.BlockSpec(memory_space=pl.ANY)],
        out_specs=pl.BlockSpec(memory_space=pl.ANY),
        scratch_shapes=[
            pltpu.VMEM((2, CPC, N), jnp.float32),
            pltpu.SemaphoreType.DMA((2,)),
            pltpu.SemaphoreType.DMA((2,)),
            pltpu.SemaphoreType.DMA((NC,)),
            pltpu.SemaphoreType.DMA((NC,)),
            pltpu.SemaphoreType.DMA((NC,)),
            pltpu.SemaphoreType.DMA((NC,)),
        ],
        compiler_params=pltpu.CompilerParams(collective_id=0),
    )(x)


# device time: 812169 ns/iter; 1.2211x vs baseline; 1.2211x over previous
import jax
import jax.numpy as jnp
from jax import lax
from jax.experimental import pallas as pl
from jax.experimental.pallas import tpu as pltpu

M_SHARD = 32768
Q_ROWS = M_SHARD // 4
N = 1024

CH = 1024
NC = Q_ROWS // CH
NH = NC // 2
CPC = 2048
NCP = M_SHARD // CPC


def kernel(x):
    m_shard, n = x.shape
    assert (m_shard, n) == (M_SHARD, N), (m_shard, n)

    def body(x_ref, out_ref, vbuf, lsems, ssems,
             xs, xr, y1s, y1r, z1s, z1r, y2s, y2r, z2s, z2r):
        my_x = lax.axis_index("x")
        my_y = lax.axis_index("y")
        my_z = lax.axis_index("z")
        xbar = 1 - my_x
        ybar = 1 - my_y
        zbar = 1 - my_z

        barrier_sem = pltpu.get_barrier_semaphore()
        for dev in [(xbar, my_y, my_z), (my_x, ybar, my_z), (my_x, my_y, zbar)]:
            pl.semaphore_signal(
                barrier_sem, inc=1,
                device_id=dev, device_id_type=pl.DeviceIdType.MESH,
            )
        pl.semaphore_wait(barrier_sem, 3)

        qi = 2 * my_y + my_z
        qi_a = 2 * ybar + my_z
        qi_b = 2 * my_y + zbar
        F = xbar * M_SHARD

        dev_x = (xbar, my_y, my_z)
        dev_a = (my_x, ybar, my_z)
        dev_b = (my_x, my_y, zbar)

        def x_rdma(c):
            return pltpu.make_async_remote_copy(
                src_ref=x_ref.at[pl.ds(qi * Q_ROWS + c * CH, CH)],
                dst_ref=out_ref.at[pl.ds(my_x * M_SHARD + qi * Q_ROWS + c * CH, CH)],
                send_sem=xs.at[c], recv_sem=xr.at[c],
                device_id=dev_x, device_id_type=pl.DeviceIdType.MESH,
            )

        def fwd(base, c, ssem, rsem, dev):
            return pltpu.make_async_remote_copy(
                src_ref=out_ref.at[pl.ds(base + c * CH, CH)],
                dst_ref=out_ref.at[pl.ds(base + c * CH, CH)],
                send_sem=ssem.at[c], recv_sem=rsem.at[c],
                device_id=dev, device_id_type=pl.DeviceIdType.MESH,
            )

        for c in range(NC):
            x_rdma(c).start()

        def load(c, slot):
            return pltpu.make_async_copy(
                x_ref.at[pl.ds(c * CPC, CPC)], vbuf.at[slot], lsems.at[slot])

        def store(c, slot):
            return pltpu.make_async_copy(
                vbuf.at[slot], out_ref.at[pl.ds(my_x * M_SHARD + c * CPC, CPC)],
                ssems.at[slot])

        load(0, 0).start()
        for c in range(NCP):
            slot = c % 2
            if c + 1 < NCP:
                if c >= 1:
                    store(c - 1, (c + 1) % 2).wait()
                load(c + 1, (c + 1) % 2).start()
            load(c, slot).wait()
            store(c, slot).start()
        for c in range(max(NCP - 2, 0), NCP):
            store(c, c % 2).wait()

        for c in range(NC):
            x_rdma(c).wait_recv()
            fwd(F + qi * Q_ROWS, c, y1s, y1r, dev_a).start()
            fwd(F + qi * Q_ROWS, c, z1s, z1r, dev_b).start()

        for c in range(NC):
            fwd(F + qi_b * Q_ROWS, c, z1s, z1r, dev_b).wait_recv()
            if c < NH:
                fwd(F + qi_b * Q_ROWS, c, y2s, y2r, dev_a).start()
            fwd(F + qi_a * Q_ROWS, c, y1s, y1r, dev_a).wait_recv()
            if c >= NH:
                fwd(F + qi_a * Q_ROWS + NH * CH, c - NH, z2s, z2r, dev_b).start()

        qi_d = 2 * ybar + zbar
        for c in range(NH):
            fwd(F + qi_d * Q_ROWS, c, y2s, y2r, dev_a).wait_recv()
            fwd(F + qi_d * Q_ROWS + NH * CH, c, z2s, z2r, dev_b).wait_recv()
        for c in range(NC):
            x_rdma(c).wait_send()
            fwd(F + qi * Q_ROWS, c, y1s, y1r, dev_a).wait_send()
            fwd(F + qi * Q_ROWS, c, z1s, z1r, dev_b).wait_send()
        for c in range(NH):
            fwd(F + qi_b * Q_ROWS, c, y2s, y2r, dev_a).wait_send()
            fwd(F + qi_a * Q_ROWS + NH * CH, c, z2s, z2r, dev_b).wait_send()

    out_shape = jax.ShapeDtypeStruct((2 * M_SHARD, N), jnp.float32)
    return pl.pallas_call(
        body,
        out_shape=out_shape,
        in_specs=[pl.BlockSpec(memory_space=pl.ANY)],
        out_specs=pl.BlockSpec(memory_space=pl.ANY),
        scratch_shapes=[
            pltpu.VMEM((2, CPC, N), jnp.float32),
            pltpu.SemaphoreType.DMA((2,)),
            pltpu.SemaphoreType.DMA((2,)),
            pltpu.SemaphoreType.DMA((NC,)),
            pltpu.SemaphoreType.DMA((NC,)),
            pltpu.SemaphoreType.DMA((NC,)),
            pltpu.SemaphoreType.DMA((NC,)),
            pltpu.SemaphoreType.DMA((NC,)),
            pltpu.SemaphoreType.DMA((NC,)),
            pltpu.SemaphoreType.DMA((NH,)),
            pltpu.SemaphoreType.DMA((NH,)),
            pltpu.SemaphoreType.DMA((NH,)),
            pltpu.SemaphoreType.DMA((NH,)),
        ],
        compiler_params=pltpu.CompilerParams(collective_id=0),
    )(x)


# device time: 715578 ns/iter; 1.3859x vs baseline; 1.1350x over previous
import jax
import jax.numpy as jnp
from jax import lax
from jax.experimental import pallas as pl
from jax.experimental.pallas import tpu as pltpu

M_SHARD = 32768
Q_ROWS = M_SHARD // 4
N = 1024

CH = 1024
NC = Q_ROWS // CH
DX = 4
DY = (4, 5)
DZ = (6, 7)
CPC = 2048
NCP = M_SHARD // CPC


def kernel(x):
    m_shard, n = x.shape
    assert (m_shard, n) == (M_SHARD, N), (m_shard, n)

    def body(x_ref, out_ref, vbuf, lsems, ssems,
             xs, xr, y1s, y1r, z1s, z1r, y2s, y2r, z2s, z2r):
        my_x = lax.axis_index("x")
        my_y = lax.axis_index("y")
        my_z = lax.axis_index("z")
        xbar = 1 - my_x
        ybar = 1 - my_y
        zbar = 1 - my_z

        barrier_sem = pltpu.get_barrier_semaphore()
        for dev in [(xbar, my_y, my_z), (my_x, ybar, my_z), (my_x, my_y, zbar)]:
            pl.semaphore_signal(
                barrier_sem, inc=1,
                device_id=dev, device_id_type=pl.DeviceIdType.MESH,
            )
        pl.semaphore_wait(barrier_sem, 3)

        qi = 2 * my_y + my_z
        qi_a = 2 * ybar + my_z
        qi_b = 2 * my_y + zbar
        qi_d = 2 * ybar + zbar
        F = xbar * M_SHARD

        dev_x = (xbar, my_y, my_z)
        dev_a = (my_x, ybar, my_z)
        dev_b = (my_x, my_y, zbar)

        def x_rdma(row, k):
            return pltpu.make_async_remote_copy(
                src_ref=x_ref.at[pl.ds(row, CH)],
                dst_ref=out_ref.at[pl.ds(my_x * M_SHARD + row, CH)],
                send_sem=xs.at[k], recv_sem=xr.at[k],
                device_id=dev_x, device_id_type=pl.DeviceIdType.MESH,
            )

        def fwd(row, k, ssem, rsem, dev):
            return pltpu.make_async_remote_copy(
                src_ref=out_ref.at[pl.ds(F + row, CH)],
                dst_ref=out_ref.at[pl.ds(F + row, CH)],
                send_sem=ssem.at[k], recv_sem=rsem.at[k],
                device_id=dev, device_id_type=pl.DeviceIdType.MESH,
            )

        for c in range(NC):
            x_rdma(qi * Q_ROWS + c * CH, c).start()
        for j in range(DX):
            x_rdma(qi_d * Q_ROWS + j * CH, NC + j).start()

        def load(c, slot):
            return pltpu.make_async_copy(
                x_ref.at[pl.ds(c * CPC, CPC)], vbuf.at[slot], lsems.at[slot])

        def store(c, slot):
            return pltpu.make_async_copy(
                vbuf.at[slot], out_ref.at[pl.ds(my_x * M_SHARD + c * CPC, CPC)],
                ssems.at[slot])

        def copy_step(c):
            slot = c % 2
            if c + 1 < NCP:
                if c >= 1:
                    store(c - 1, (c + 1) % 2).wait()
                load(c + 1, (c + 1) % 2).start()
            load(c, slot).wait()
            store(c, slot).start()

        load(0, 0).start()

        for c in range(NC):
            copy_step(2 * c)
            copy_step(2 * c + 1)
            x_rdma(qi * Q_ROWS + c * CH, c).wait_recv()
            fwd(qi * Q_ROWS + c * CH, c, y1s, y1r, dev_a).start()
            fwd(qi * Q_ROWS + c * CH, c, z1s, z1r, dev_b).start()
        for c in range(max(NCP - 2, 0), NCP):
            store(c, c % 2).wait()

        for c in range(NC):
            fwd(qi_b * Q_ROWS + c * CH, c, z1s, z1r, dev_b).wait_recv()
            if c in DY:
                fwd(qi_b * Q_ROWS + c * CH, c - DY[0], y2s, y2r, dev_a).start()
            fwd(qi_a * Q_ROWS + c * CH, c, y1s, y1r, dev_a).wait_recv()
            if c in DZ:
                fwd(qi_a * Q_ROWS + c * CH, c - DZ[0], z2s, z2r, dev_b).start()

        for j in range(DX):
            x_rdma(qi_d * Q_ROWS + j * CH, NC + j).wait_recv()
        for j, c in enumerate(DY):
            fwd(qi_d * Q_ROWS + c * CH, j, y2s, y2r, dev_a).wait_recv()
        for j, c in enumerate(DZ):
            fwd(qi_d * Q_ROWS + c * CH, j, z2s, z2r, dev_b).wait_recv()
        for c in range(NC):
            x_rdma(qi * Q_ROWS + c * CH, c).wait_send()
            fwd(qi * Q_ROWS + c * CH, c, y1s, y1r, dev_a).wait_send()
            fwd(qi * Q_ROWS + c * CH, c, z1s, z1r, dev_b).wait_send()
        for j in range(DX):
            x_rdma(qi_d * Q_ROWS + j * CH, NC + j).wait_send()
        for j, c in enumerate(DY):
            fwd(qi_b * Q_ROWS + c * CH, j, y2s, y2r, dev_a).wait_send()
        for j, c in enumerate(DZ):
            fwd(qi_a * Q_ROWS + c * CH, j, z2s, z2r, dev_b).wait_send()

    out_shape = jax.ShapeDtypeStruct((2 * M_SHARD, N), jnp.float32)
    return pl.pallas_call(
        body,
        out_shape=out_shape,
        in_specs=[pl.BlockSpec(memory_space=pl.ANY)],
        out_specs=pl.BlockSpec(memory_space=pl.ANY),
        scratch_shapes=[
            pltpu.VMEM((2, CPC, N), jnp.float32),
            pltpu.SemaphoreType.DMA((2,)),
            pltpu.SemaphoreType.DMA((2,)),
            pltpu.SemaphoreType.DMA((NC + DX,)),
            pltpu.SemaphoreType.DMA((NC + DX,)),
            pltpu.SemaphoreType.DMA((NC,)),
            pltpu.SemaphoreType.DMA((NC,)),
            pltpu.SemaphoreType.DMA((NC,)),
            pltpu.SemaphoreType.DMA((NC,)),
            pltpu.SemaphoreType.DMA((len(DY),)),
            pltpu.SemaphoreType.DMA((len(DY),)),
            pltpu.SemaphoreType.DMA((len(DZ),)),
            pltpu.SemaphoreType.DMA((len(DZ),)),
        ],
        compiler_params=pltpu.CompilerParams(collective_id=0),
    )(x)


# device time: 697570 ns/iter; 1.4217x vs baseline; 1.0258x over previous
import jax
import jax.numpy as jnp
from jax import lax
from jax.experimental import pallas as pl
from jax.experimental.pallas import tpu as pltpu

M_SHARD = 32768
Q_ROWS = M_SHARD // 4
N = 1024

CH = 512
NC = Q_ROWS // CH
DX = 7
DY = (7, 8, 9, 10, 11)
DZ = (12, 13, 14, 15)
CPC = 2048
NCP = M_SHARD // CPC


def kernel(x):
    m_shard, n = x.shape
    assert (m_shard, n) == (M_SHARD, N), (m_shard, n)

    def body(x_ref, out_ref, vbuf, lsems, ssems,
             xs, xr, y1s, y1r, z1s, z1r, y2s, y2r, z2s, z2r):
        my_x = lax.axis_index("x")
        my_y = lax.axis_index("y")
        my_z = lax.axis_index("z")
        xbar = 1 - my_x
        ybar = 1 - my_y
        zbar = 1 - my_z

        barrier_sem = pltpu.get_barrier_semaphore()
        for dev in [(xbar, my_y, my_z), (my_x, ybar, my_z), (my_x, my_y, zbar)]:
            pl.semaphore_signal(
                barrier_sem, inc=1,
                device_id=dev, device_id_type=pl.DeviceIdType.MESH,
            )
        pl.semaphore_wait(barrier_sem, 3)

        qi = 2 * my_y + my_z
        qi_a = 2 * ybar + my_z
        qi_b = 2 * my_y + zbar
        qi_d = 2 * ybar + zbar
        F = xbar * M_SHARD

        dev_x = (xbar, my_y, my_z)
        dev_a = (my_x, ybar, my_z)
        dev_b = (my_x, my_y, zbar)

        def x_rdma(row, k):
            return pltpu.make_async_remote_copy(
                src_ref=x_ref.at[pl.ds(row, CH)],
                dst_ref=out_ref.at[pl.ds(my_x * M_SHARD + row, CH)],
                send_sem=xs.at[k], recv_sem=xr.at[k],
                device_id=dev_x, device_id_type=pl.DeviceIdType.MESH,
            )

        def fwd(row, k, ssem, rsem, dev):
            return pltpu.make_async_remote_copy(
                src_ref=out_ref.at[pl.ds(F + row, CH)],
                dst_ref=out_ref.at[pl.ds(F + row, CH)],
                send_sem=ssem.at[k], recv_sem=rsem.at[k],
                device_id=dev, device_id_type=pl.DeviceIdType.MESH,
            )

        for c in range(NC):
            x_rdma(qi * Q_ROWS + c * CH, c).start()
        for j in range(DX):
            x_rdma(qi_d * Q_ROWS + j * CH, NC + j).start()

        def load(c, slot):
            return pltpu.make_async_copy(
                x_ref.at[pl.ds(c * CPC, CPC)], vbuf.at[slot], lsems.at[slot])

        def store(c, slot):
            return pltpu.make_async_copy(
                vbuf.at[slot], out_ref.at[pl.ds(my_x * M_SHARD + c * CPC, CPC)],
                ssems.at[slot])

        def copy_step(c):
            slot = c % 2
            if c + 1 < NCP:
                if c >= 1:
                    store(c - 1, (c + 1) % 2).wait()
                load(c + 1, (c + 1) % 2).start()
            load(c, slot).wait()
            store(c, slot).start()

        load(0, 0).start()

        for c in range(NC):
            copy_step(c)
            x_rdma(qi * Q_ROWS + c * CH, c).wait_recv()
            fwd(qi * Q_ROWS + c * CH, c, y1s, y1r, dev_a).start()
            fwd(qi * Q_ROWS + c * CH, c, z1s, z1r, dev_b).start()
        for c in range(max(NCP - 2, 0), NCP):
            store(c, c % 2).wait()

        for c in range(NC):
            fwd(qi_b * Q_ROWS + c * CH, c, z1s, z1r, dev_b).wait_recv()
            if c in DY:
                fwd(qi_b * Q_ROWS + c * CH, c - DY[0], y2s, y2r, dev_a).start()
            fwd(qi_a * Q_ROWS + c * CH, c, y1s, y1r, dev_a).wait_recv()
            if c in DZ:
                fwd(qi_a * Q_ROWS + c * CH, c - DZ[0], z2s, z2r, dev_b).start()

        for j in range(DX):
            x_rdma(qi_d * Q_ROWS + j * CH, NC + j).wait_recv()
        for j, c in enumerate(DY):
            fwd(qi_d * Q_ROWS + c * CH, j, y2s, y2r, dev_a).wait_recv()
        for j, c in enumerate(DZ):
            fwd(qi_d * Q_ROWS + c * CH, j, z2s, z2r, dev_b).wait_recv()
        for c in range(NC):
            x_rdma(qi * Q_ROWS + c * CH, c).wait_send()
            fwd(qi * Q_ROWS + c * CH, c, y1s, y1r, dev_a).wait_send()
            fwd(qi * Q_ROWS + c * CH, c, z1s, z1r, dev_b).wait_send()
        for j in range(DX):
            x_rdma(qi_d * Q_ROWS + j * CH, NC + j).wait_send()
        for j, c in enumerate(DY):
            fwd(qi_b * Q_ROWS + c * CH, j, y2s, y2r, dev_a).wait_send()
        for j, c in enumerate(DZ):
            fwd(qi_a * Q_ROWS + c * CH, j, z2s, z2r, dev_b).wait_send()

    out_shape = jax.ShapeDtypeStruct((2 * M_SHARD, N), jnp.float32)
    return pl.pallas_call(
        body,
        out_shape=out_shape,
        in_specs=[pl.BlockSpec(memory_space=pl.ANY)],
        out_specs=pl.BlockSpec(memory_space=pl.ANY),
        scratch_shapes=[
            pltpu.VMEM((2, CPC, N), jnp.float32),
            pltpu.SemaphoreType.DMA((2,)),
            pltpu.SemaphoreType.DMA((2,)),
            pltpu.SemaphoreType.DMA((NC + DX,)),
            pltpu.SemaphoreType.DMA((NC + DX,)),
            pltpu.SemaphoreType.DMA((NC,)),
            pltpu.SemaphoreType.DMA((NC,)),
            pltpu.SemaphoreType.DMA((NC,)),
            pltpu.SemaphoreType.DMA((NC,)),
            pltpu.SemaphoreType.DMA((len(DY),)),
            pltpu.SemaphoreType.DMA((len(DY),)),
            pltpu.SemaphoreType.DMA((len(DZ),)),
            pltpu.SemaphoreType.DMA((len(DZ),)),
        ],
        compiler_params=pltpu.CompilerParams(collective_id=0),
    )(x)
